# baseline (speedup 1.0000x reference)
import functools

import jax
import jax.numpy as jnp
from jax.experimental import pallas as pl
from jax.experimental.pallas import tpu as pltpu

_EPS = 1e-5


def _prep_kernel(bt_ref, g0_ref, b0_ref, emb_ref, w1b_ref, stats_ref, wef_ref):
    b12_t = bt_ref[1:3, :]
    b34_t = bt_ref[3:5, :]
    wh_t = b34_t - b12_t + 1.0
    c_t = b12_t + 0.5 * wh_t
    cs_t = jnp.concatenate([c_t, wh_t], axis=0)
    mu = jnp.mean(cs_t, axis=1, keepdims=True)
    var = jnp.mean((cs_t - mu) ** 2, axis=1, keepdims=True)
    scale = g0_ref[...] * jax.lax.rsqrt(var + _EPS)
    shift = b0_ref[...] - mu * scale
    stats_ref[...] = jnp.transpose(jnp.concatenate([scale, shift], axis=1))
    wef = jnp.dot(emb_ref[...].astype(jnp.bfloat16),
                  w1b_ref[...].astype(jnp.bfloat16),
                  preferred_element_type=jnp.float32)
    wef_ref[...] = wef.astype(jnp.bfloat16)


def _fused_kernel(feat_ref, dist_ref, box_ref, stats_ref, wpos_ref, bpos_ref,
                  wef_ref, w1a_ref, w1c_ref, b1_ref, g1_ref, b1n_ref, w2_ref,
                  b2_ref, out_ref, h_scr, sums_scr, *, inv_n, tile):
    p = pl.program_id(0)
    i = pl.program_id(1)

    @pl.when(p == 0)
    def _phase0():
        h = jnp.dot(feat_ref[...].astype(jnp.bfloat16), w1a_ref[...],
                    preferred_element_type=jnp.float32)

        x = box_ref[...]
        b12 = x[:, 1:3]
        b34 = x[:, 3:5]
        wh = b34 - b12 + 1.0
        c = b12 + 0.5 * wh
        cs = jnp.concatenate([c, wh], axis=1)
        csn = cs * stats_ref[0:1, :] + stats_ref[1:2, :]
        pe = jnp.dot(csn.astype(jnp.bfloat16), wpos_ref[...],
                     preferred_element_type=jnp.float32) + bpos_ref[...]
        pe = jnp.maximum(pe, 0.0)

        h2 = h + jnp.dot(dist_ref[...].astype(jnp.bfloat16), wef_ref[...],
                         preferred_element_type=jnp.float32)
        h2 = h2 + jnp.dot(pe.astype(jnp.bfloat16), w1c_ref[...],
                          preferred_element_type=jnp.float32)
        h2 = h2 + b1_ref[...]
        h_scr[pl.ds(i * tile, tile), :] = h2.astype(jnp.bfloat16)

        part = jnp.concatenate([jnp.sum(h2, axis=0, keepdims=True),
                                jnp.sum(h2 * h2, axis=0, keepdims=True)],
                               axis=0)

        @pl.when(i == 0)
        def _():
            sums_scr[...] = part

        @pl.when(i != 0)
        def _():
            sums_scr[...] += part

    @pl.when(p == 1)
    def _phase1():
        mean = sums_scr[0:1, :] * inv_n
        ex2 = sums_scr[1:2, :] * inv_n
        var = ex2 - mean * mean
        scale_f = g1_ref[...] * jax.lax.rsqrt(var + _EPS)
        scale = scale_f.astype(jnp.bfloat16)
        shift = (b1n_ref[...] - mean * scale_f).astype(jnp.bfloat16)
        hb = h_scr[pl.ds(i * tile, tile), :]
        a = jnp.maximum(hb * scale + shift, jnp.bfloat16(0.0))
        out_ref[...] = jnp.dot(a, w2_ref[...],
                               preferred_element_type=jnp.float32) + b2_ref[...]


def kernel(features, distribution, boxes, obj_embed_w, bn0_g, bn0_b,
           w_pos, b_pos, w1, b1, bn1_g, bn1_b, w2, b2):
    n, obj_dim = features.shape
    nc1 = distribution.shape[1]
    emb_dim = obj_embed_w.shape[1]
    pos_dim = w_pos.shape[1]
    hid = w1.shape[1]
    n_out = w2.shape[1]
    f32 = jnp.float32
    bf16 = jnp.bfloat16

    bt = jnp.transpose(boxes)
    g0 = bn0_g.reshape(4, 1).astype(f32)
    b0 = bn0_b.reshape(4, 1).astype(f32)
    w1a = w1[:obj_dim].astype(bf16)
    w1b = w1[obj_dim:obj_dim + emb_dim]
    w1c = w1[obj_dim + emb_dim:].astype(bf16)
    wpos = w_pos.astype(bf16)
    bpos = b_pos.reshape(1, pos_dim)
    b1r = b1.reshape(1, hid)
    g1 = bn1_g.reshape(1, hid)
    b1n = bn1_b.reshape(1, hid)
    w2b = w2.astype(bf16)
    b2r = b2.reshape(1, n_out)

    stats, wef = pl.pallas_call(
        _prep_kernel,
        grid=(1,),
        in_specs=[
            pl.BlockSpec((5, n), lambda i: (0, 0)),
            pl.BlockSpec((4, 1), lambda i: (0, 0)),
            pl.BlockSpec((4, 1), lambda i: (0, 0)),
            pl.BlockSpec((nc1, emb_dim), lambda i: (0, 0)),
            pl.BlockSpec((emb_dim, hid), lambda i: (0, 0)),
        ],
        out_specs=[
            pl.BlockSpec((2, 4), lambda i: (0, 0)),
            pl.BlockSpec((nc1, hid), lambda i: (0, 0)),
        ],
        out_shape=[
            jax.ShapeDtypeStruct((2, 4), f32),
            jax.ShapeDtypeStruct((nc1, hid), bf16),
        ],
    )(bt, g0, b0, obj_embed_w, w1b)

    tile = 400
    nt = n // tile

    def _row_map(p, i):
        return (jnp.where(p == 0, i, nt - 1), 0)

    def _out_map(p, i):
        return (jnp.where(p == 1, i, 0), 0)

    _const = lambda p, i: (0, 0)

    logits = pl.pallas_call(
        functools.partial(_fused_kernel, inv_n=1.0 / n, tile=tile),
        grid=(2, nt),
        in_specs=[
            pl.BlockSpec((tile, obj_dim), _row_map),
            pl.BlockSpec((tile, nc1), _row_map),
            pl.BlockSpec((tile, 5), _row_map),
            pl.BlockSpec((2, 4), _const),
            pl.BlockSpec((4, pos_dim), _const),
            pl.BlockSpec((1, pos_dim), _const),
            pl.BlockSpec((nc1, hid), _const),
            pl.BlockSpec((obj_dim, hid), _const),
            pl.BlockSpec((pos_dim, hid), _const),
            pl.BlockSpec((1, hid), _const),
            pl.BlockSpec((1, hid), _const),
            pl.BlockSpec((1, hid), _const),
            pl.BlockSpec((hid, n_out), _const),
            pl.BlockSpec((1, n_out), _const),
        ],
        out_specs=pl.BlockSpec((tile, n_out), _out_map),
        out_shape=jax.ShapeDtypeStruct((n, n_out), f32),
        scratch_shapes=[
            pltpu.VMEM((n, hid), bf16),
            pltpu.VMEM((2, hid), f32),
        ],
    )(features, distribution, boxes, stats, wpos, bpos, wef, w1a, w1c, b1r,
      g1, b1n, w2b, b2r)

    return logits


# speedup vs baseline: 1.0011x; 1.0011x over previous
import functools

import jax
import jax.numpy as jnp
from jax.experimental import pallas as pl
from jax.experimental.pallas import tpu as pltpu

_EPS = 1e-5


def _prep_kernel(bt_ref, g0_ref, b0_ref, emb_ref, w1b_ref, stats_ref, wef_ref):
    b12_t = bt_ref[1:3, :]
    b34_t = bt_ref[3:5, :]
    wh_t = b34_t - b12_t + 1.0
    c_t = b12_t + 0.5 * wh_t
    cs_t = jnp.concatenate([c_t, wh_t], axis=0)
    mu = jnp.mean(cs_t, axis=1, keepdims=True)
    var = jnp.mean((cs_t - mu) ** 2, axis=1, keepdims=True)
    scale = g0_ref[...] * jax.lax.rsqrt(var + _EPS)
    shift = b0_ref[...] - mu * scale
    stats_ref[...] = jnp.transpose(jnp.concatenate([scale, shift], axis=1))
    wef_ref[...] = jnp.dot(emb_ref[...], w1b_ref[...],
                           precision=jax.lax.Precision.DEFAULT,
                           preferred_element_type=jnp.float32)


def _fused_kernel(feat_hbm, dist_ref, box_ref, stats_ref, wpos_ref, bpos_ref,
                  wef_ref, w1a_ref, w1c_ref, b1_ref, g1_ref, b1n_ref, w2_ref,
                  b2_ref, out_ref, h_scr, sums_scr, fb0, fb1, sm0, sm1,
                  *, inv_n, tile, half):
    p = pl.program_id(0)
    i = pl.program_id(1)
    nt = pl.num_programs(1)

    @pl.when(p == 0)
    def _phase0():
        @pl.when(i == 0)
        def _():
            pltpu.make_async_copy(feat_hbm.at[pl.ds(0, half)], fb0, sm0).start()
            pltpu.make_async_copy(feat_hbm.at[pl.ds(half, half)], fb1, sm1).start()

        x = box_ref[...]
        b12 = x[:, 1:3]
        b34 = x[:, 3:5]
        wh = b34 - b12 + 1.0
        c = b12 + 0.5 * wh
        cs = jnp.concatenate([c, wh], axis=1)
        csn = cs * stats_ref[0:1, :] + stats_ref[1:2, :]
        pe = jnp.dot(csn, wpos_ref[...],
                     precision=jax.lax.Precision.DEFAULT,
                     preferred_element_type=jnp.float32) + bpos_ref[...]
        pe = jnp.maximum(pe, 0.0)
        rest = jnp.dot(dist_ref[...], wef_ref[...],
                       precision=jax.lax.Precision.DEFAULT,
                       preferred_element_type=jnp.float32)
        rest = rest + jnp.dot(pe, w1c_ref[...],
                              precision=jax.lax.Precision.DEFAULT,
                              preferred_element_type=jnp.float32)
        rest = rest + b1_ref[...]

        pltpu.make_async_copy(feat_hbm.at[pl.ds(i * tile, half)],
                              fb0, sm0).wait()
        ha = jnp.dot(fb0[...].astype(jnp.bfloat16), w1a_ref[...],
                     preferred_element_type=jnp.float32) + rest[:half]

        @pl.when(i + 1 < nt)
        def _():
            pltpu.make_async_copy(feat_hbm.at[pl.ds((i + 1) * tile, half)],
                                  fb0, sm0).start()

        pltpu.make_async_copy(feat_hbm.at[pl.ds(i * tile + half, half)],
                              fb1, sm1).wait()
        hb = jnp.dot(fb1[...].astype(jnp.bfloat16), w1a_ref[...],
                     preferred_element_type=jnp.float32) + rest[half:]

        @pl.when(i + 1 < nt)
        def _():
            pltpu.make_async_copy(
                feat_hbm.at[pl.ds((i + 1) * tile + half, half)],
                fb1, sm1).start()

        h2 = jnp.concatenate([ha, hb], axis=0)
        h_scr[pl.ds(i * tile, tile), :] = h2.astype(jnp.bfloat16)

        part = jnp.concatenate([jnp.sum(h2, axis=0, keepdims=True),
                                jnp.sum(h2 * h2, axis=0, keepdims=True)],
                               axis=0)

        @pl.when(i == 0)
        def _():
            sums_scr[...] = part

        @pl.when(i != 0)
        def _():
            sums_scr[...] += part

    @pl.when(p == 1)
    def _phase1():
        mean = sums_scr[0:1, :] * inv_n
        ex2 = sums_scr[1:2, :] * inv_n
        var = ex2 - mean * mean
        scale_f = g1_ref[...] * jax.lax.rsqrt(var + _EPS)
        scale = scale_f.astype(jnp.bfloat16)
        shift = (b1n_ref[...] - mean * scale_f).astype(jnp.bfloat16)
        hbk = h_scr[pl.ds(i * tile, tile), :]
        a = jnp.maximum(hbk * scale + shift, jnp.bfloat16(0.0))
        out_ref[...] = jnp.dot(a, w2_ref[...],
                               preferred_element_type=jnp.float32) + b2_ref[...]


def kernel(features, distribution, boxes, obj_embed_w, bn0_g, bn0_b,
           w_pos, b_pos, w1, b1, bn1_g, bn1_b, w2, b2):
    n, obj_dim = features.shape
    nc1 = distribution.shape[1]
    emb_dim = obj_embed_w.shape[1]
    pos_dim = w_pos.shape[1]
    hid = w1.shape[1]
    n_out = w2.shape[1]
    f32 = jnp.float32
    bf16 = jnp.bfloat16

    bt = jnp.transpose(boxes)
    g0 = bn0_g.reshape(4, 1).astype(f32)
    b0 = bn0_b.reshape(4, 1).astype(f32)
    w1a = w1[:obj_dim].astype(bf16)
    w1b = w1[obj_dim:obj_dim + emb_dim]
    w1c = w1[obj_dim + emb_dim:]
    bpos = b_pos.reshape(1, pos_dim)
    b1r = b1.reshape(1, hid)
    g1 = bn1_g.reshape(1, hid)
    b1n = bn1_b.reshape(1, hid)
    w2b = w2.astype(bf16)
    b2r = b2.reshape(1, n_out)

    stats, wef = pl.pallas_call(
        _prep_kernel,
        grid=(1,),
        in_specs=[
            pl.BlockSpec((5, n), lambda i: (0, 0)),
            pl.BlockSpec((4, 1), lambda i: (0, 0)),
            pl.BlockSpec((4, 1), lambda i: (0, 0)),
            pl.BlockSpec((nc1, emb_dim), lambda i: (0, 0)),
            pl.BlockSpec((emb_dim, hid), lambda i: (0, 0)),
        ],
        out_specs=[
            pl.BlockSpec((2, 4), lambda i: (0, 0)),
            pl.BlockSpec((nc1, hid), lambda i: (0, 0)),
        ],
        out_shape=[
            jax.ShapeDtypeStruct((2, 4), f32),
            jax.ShapeDtypeStruct((nc1, hid), f32),
        ],
    )(bt, g0, b0, obj_embed_w, w1b)

    tile = 800
    half = tile // 2
    nt = n // tile

    def _row_map(p, i):
        return (jnp.where(p == 0, i, nt - 1), 0)

    def _out_map(p, i):
        return (jnp.where(p == 1, i, 0), 0)

    _const = lambda p, i: (0, 0)

    logits = pl.pallas_call(
        functools.partial(_fused_kernel, inv_n=1.0 / n, tile=tile, half=half),
        grid=(2, nt),
        in_specs=[
            pl.BlockSpec(memory_space=pl.ANY),
            pl.BlockSpec((tile, nc1), _row_map),
            pl.BlockSpec((tile, 5), _row_map),
            pl.BlockSpec((2, 4), _const),
            pl.BlockSpec((4, pos_dim), _const),
            pl.BlockSpec((1, pos_dim), _const),
            pl.BlockSpec((nc1, hid), _const),
            pl.BlockSpec((obj_dim, hid), _const),
            pl.BlockSpec((pos_dim, hid), _const),
            pl.BlockSpec((1, hid), _const),
            pl.BlockSpec((1, hid), _const),
            pl.BlockSpec((1, hid), _const),
            pl.BlockSpec((hid, n_out), _const),
            pl.BlockSpec((1, n_out), _const),
        ],
        out_specs=pl.BlockSpec((tile, n_out), _out_map),
        out_shape=jax.ShapeDtypeStruct((n, n_out), f32),
        scratch_shapes=[
            pltpu.VMEM((n, hid), bf16),
            pltpu.VMEM((2, hid), f32),
            pltpu.VMEM((half, obj_dim), f32),
            pltpu.VMEM((half, obj_dim), f32),
            pltpu.SemaphoreType.DMA,
            pltpu.SemaphoreType.DMA,
        ],
    )(features, distribution, boxes, stats, w_pos, bpos, wef, w1a, w1c, b1r,
      g1, b1n, w2b, b2r)

    return logits
